# Initial kernel scaffold; baseline (speedup 1.0000x reference)
import jax
import jax.numpy as jnp
from jax.experimental import pallas as pl


def kernel(x, indices):
    raise NotImplementedError("write your pallas kernel here")


# speedup vs baseline: 7.6003x; 7.6003x over previous
import jax
import jax.numpy as jnp
from jax.experimental import pallas as pl
from jax.experimental.pallas import tpu as pltpu

B, T, C = 64, 577, 768
ABLATION_VALUE = 10000000.0
INF = float("inf")


def _mins_kernel(idx_ref, x_ref, v_ref, fb_s, mb_s, sfb_s, v_s):
    j = pl.program_id(0)
    xb = x_ref[0]
    rowmins = jnp.min(xb, axis=1, keepdims=True)
    fb = jnp.min(rowmins)
    idx = idx_ref[j]
    tids = jax.lax.broadcasted_iota(jnp.int32, (T, 1), 0)
    mb = jnp.min(jnp.where(tids == idx, INF, rowmins))
    fb_s[j] = fb
    mb_s[j] = mb

    @pl.when(j == B - 1)
    def _():
        def bwd(t, carry):
            i = B - 1 - t
            carry = jnp.minimum(carry, fb_s[i])
            sfb_s[i] = carry
            return carry

        jax.lax.fori_loop(0, B, bwd, jnp.float32(INF))

        def fwd(i, carry):
            pmb, vmin = carry
            m = jnp.minimum(jnp.minimum(pmb, sfb_s[i]), vmin)
            v = jnp.where(m == 0.0, jnp.float32(0.0), m - ABLATION_VALUE)
            v_s[i] = v
            return jnp.minimum(pmb, mb_s[i]), jnp.minimum(vmin, v)

        jax.lax.fori_loop(0, B, fwd, (jnp.float32(INF), jnp.float32(INF)))

        def wr(i, _):
            v_ref[pl.ds(i, 1), :] = jnp.full((1, 128), v_s[i], jnp.float32)
            return 0

        jax.lax.fori_loop(0, B, wr, 0)


def _transpose_kernel(idx_ref, v_ref, x_ref, out_ref):
    j = pl.program_id(0)
    xt = x_ref[0].T
    idx = idx_ref[j]
    val = v_ref[j]
    tcol = jax.lax.broadcasted_iota(jnp.int32, (C, T), 1)
    out_ref[0] = jnp.where(tcol == idx, val, xt)


def kernel(x, indices):
    v_pad = pl.pallas_call(
        _mins_kernel,
        grid_spec=pltpu.PrefetchScalarGridSpec(
            num_scalar_prefetch=1,
            grid=(B,),
            in_specs=[pl.BlockSpec((1, T, C), lambda j, idx_ref: (j, 0, 0))],
            out_specs=pl.BlockSpec((B, 128), lambda j, idx_ref: (0, 0)),
            scratch_shapes=[
                pltpu.SMEM((B,), jnp.float32),
                pltpu.SMEM((B,), jnp.float32),
                pltpu.SMEM((B,), jnp.float32),
                pltpu.SMEM((B,), jnp.float32),
            ],
        ),
        out_shape=jax.ShapeDtypeStruct((B, 128), jnp.float32),
    )(indices, x)
    v = v_pad[:, 0]

    out = pl.pallas_call(
        _transpose_kernel,
        grid_spec=pltpu.PrefetchScalarGridSpec(
            num_scalar_prefetch=2,
            grid=(B,),
            in_specs=[pl.BlockSpec((1, T, C), lambda j, *_: (j, 0, 0))],
            out_specs=pl.BlockSpec((1, C, T), lambda j, *_: (j, 0, 0)),
        ),
        out_shape=jax.ShapeDtypeStruct((B, C, T), jnp.float32),
    )(indices, v, x)
    return out


# speedup vs baseline: 8.5769x; 1.1285x over previous
import jax
import jax.numpy as jnp
from jax.experimental import pallas as pl
from jax.experimental.pallas import tpu as pltpu

B, T, C = 64, 577, 768
ABLATION_VALUE = 10000000.0
INF = float("inf")
BB = 8
BT = 4


def _mins_kernel(idx_ref, x_ref, v_ref, fb_s, mb_s, sfb_s, v_s):
    j = pl.program_id(0)
    xb = x_ref[...]
    rowmins = jnp.min(xb, axis=2)
    tids = jax.lax.broadcasted_iota(jnp.int32, (BB, T), 1)
    for k in range(BB):
        b = j * BB + k
        idx = idx_ref[b]
        fb_s[b] = jnp.min(rowmins[k])
        mb_s[b] = jnp.min(jnp.where(tids[k] == idx, INF, rowmins[k]))

    @pl.when(j == (B // BB) - 1)
    def _():
        def bwd(t, carry):
            i = B - 1 - t
            carry = jnp.minimum(carry, fb_s[i])
            sfb_s[i] = carry
            return carry

        jax.lax.fori_loop(0, B, bwd, jnp.float32(INF))

        def fwd(i, carry):
            pmb, vmin = carry
            m = jnp.minimum(jnp.minimum(pmb, sfb_s[i]), vmin)
            v = jnp.where(m == 0.0, jnp.float32(0.0), m - ABLATION_VALUE)
            v_s[i] = v
            return jnp.minimum(pmb, mb_s[i]), jnp.minimum(vmin, v)

        jax.lax.fori_loop(0, B, fwd, (jnp.float32(INF), jnp.float32(INF)))

        def wr(i, _):
            v_ref[pl.ds(i, 1), :] = jnp.full((1, 128), v_s[i], jnp.float32)
            return 0

        jax.lax.fori_loop(0, B, wr, 0)


def _transpose_kernel(idx_ref, v_ref, x_ref, out_ref):
    j = pl.program_id(0)
    xt = jnp.transpose(x_ref[...], (0, 2, 1))
    tcol = jax.lax.broadcasted_iota(jnp.int32, (BT, C, T), 2)
    idxs = jnp.concatenate(
        [jnp.full((1, 1, 1), idx_ref[j * BT + k], jnp.int32) for k in range(BT)], 0
    )
    vals = jnp.concatenate(
        [jnp.full((1, 1, 1), v_ref[j * BT + k], jnp.float32) for k in range(BT)], 0
    )
    out_ref[...] = jnp.where(tcol == idxs, vals, xt)


def kernel(x, indices):
    v_pad = pl.pallas_call(
        _mins_kernel,
        grid_spec=pltpu.PrefetchScalarGridSpec(
            num_scalar_prefetch=1,
            grid=(B // BB,),
            in_specs=[pl.BlockSpec((BB, T, C), lambda j, idx_ref: (j, 0, 0))],
            out_specs=pl.BlockSpec((B, 128), lambda j, idx_ref: (0, 0)),
            scratch_shapes=[
                pltpu.SMEM((B,), jnp.float32),
                pltpu.SMEM((B,), jnp.float32),
                pltpu.SMEM((B,), jnp.float32),
                pltpu.SMEM((B,), jnp.float32),
            ],
        ),
        out_shape=jax.ShapeDtypeStruct((B, 128), jnp.float32),
    )(indices, x)
    v = v_pad[:, 0]

    out = pl.pallas_call(
        _transpose_kernel,
        grid_spec=pltpu.PrefetchScalarGridSpec(
            num_scalar_prefetch=2,
            grid=(B // BT,),
            in_specs=[pl.BlockSpec((BT, T, C), lambda j, *_: (j, 0, 0))],
            out_specs=pl.BlockSpec((BT, C, T), lambda j, *_: (j, 0, 0)),
        ),
        out_shape=jax.ShapeDtypeStruct((B, C, T), jnp.float32),
    )(indices, v, x)
    return out


# speedup vs baseline: 9.2273x; 1.0758x over previous
import jax
import jax.numpy as jnp
from jax.experimental import pallas as pl
from jax.experimental.pallas import tpu as pltpu

B, T, C = 64, 577, 768
ABLATION_VALUE = 10000000.0
INF = float("inf")
NS = 4
BB = 4


def _mins_kernel(idx_ref, *refs):
    x_refs = refs[:NS]
    v_ref = refs[NS]
    fb_s, mb_s, sfb_s, v_s = refs[NS + 1:]
    j = pl.program_id(0)
    for s in range(NS):
        xb = x_refs[s][...]
        rowmins = jnp.min(xb, axis=2)
        tids = jax.lax.broadcasted_iota(jnp.int32, (BB, T), 1)
        for k in range(BB):
            b = (j * NS + s) * BB + k
            idx = idx_ref[b]
            fb_s[b] = jnp.min(rowmins[k])
            mb_s[b] = jnp.min(jnp.where(tids[k] == idx, INF, rowmins[k]))

    @pl.when(j == pl.num_programs(0) - 1)
    def _():
        def bwd(t, carry):
            i = B - 1 - t
            carry = jnp.minimum(carry, fb_s[i])
            sfb_s[i] = carry
            return carry

        jax.lax.fori_loop(0, B, bwd, jnp.float32(INF))

        def fwd(i, carry):
            pmb, vmin = carry
            m = jnp.minimum(jnp.minimum(pmb, sfb_s[i]), vmin)
            v = jnp.where(m == 0.0, jnp.float32(0.0), m - ABLATION_VALUE)
            v_s[i] = v
            return jnp.minimum(pmb, mb_s[i]), jnp.minimum(vmin, v)

        jax.lax.fori_loop(0, B, fwd, (jnp.float32(INF), jnp.float32(INF)))

        def wr(i, _):
            v_ref[pl.ds(i, 1), :] = jnp.full((1, 128), v_s[i], jnp.float32)
            return 0

        jax.lax.fori_loop(0, B, wr, 0)


def _transpose_kernel(idx_ref, v_ref, *refs):
    x_refs = refs[:NS]
    out_ref = refs[NS]
    j = pl.program_id(0)
    for s in range(NS):
        b = j * NS + s
        xt = x_refs[s][0].T
        tcol = jax.lax.broadcasted_iota(jnp.int32, (C, T), 1)
        out_ref[s] = jnp.where(tcol == idx_ref[b], v_ref[b], xt)


def kernel(x, indices):
    n_mins_steps = B // (NS * BB)
    v_pad = pl.pallas_call(
        _mins_kernel,
        grid_spec=pltpu.PrefetchScalarGridSpec(
            num_scalar_prefetch=1,
            grid=(n_mins_steps,),
            in_specs=[
                pl.BlockSpec(
                    (BB, T, C),
                    (lambda s: (lambda j, idx_ref: (j * NS + s, 0, 0)))(s),
                )
                for s in range(NS)
            ],
            out_specs=pl.BlockSpec((B, 128), lambda j, idx_ref: (0, 0)),
            scratch_shapes=[
                pltpu.SMEM((B,), jnp.float32),
                pltpu.SMEM((B,), jnp.float32),
                pltpu.SMEM((B,), jnp.float32),
                pltpu.SMEM((B,), jnp.float32),
            ],
        ),
        out_shape=jax.ShapeDtypeStruct((B, 128), jnp.float32),
    )(indices, *([x] * NS))
    v = v_pad[:, 0]

    n_tr_steps = B // NS
    out = pl.pallas_call(
        _transpose_kernel,
        grid_spec=pltpu.PrefetchScalarGridSpec(
            num_scalar_prefetch=2,
            grid=(n_tr_steps,),
            in_specs=[
                pl.BlockSpec(
                    (1, T, C),
                    (lambda s: (lambda j, *_: (j * NS + s, 0, 0)))(s),
                )
                for s in range(NS)
            ],
            out_specs=pl.BlockSpec((NS, C, T), lambda j, *_: (j, 0, 0)),
        ),
        out_shape=jax.ShapeDtypeStruct((B, C, T), jnp.float32),
    )(indices, v, *([x] * NS))
    return out
